# speedup vs baseline: 2.1730x; 2.1730x over previous
import functools

import jax
import jax.numpy as jnp
from jax import lax
from jax.experimental import pallas as pl
from jax.experimental.pallas import tpu as pltpu
from jax.experimental.pallas import tpu_sc as plsc

NUM_CORES = 2
NUM_SUBCORES = 16
NW = NUM_CORES * NUM_SUBCORES
B = 16384
D = 128
CHUNK = 128
CPW = B // (NW * CHUNK)


def _transe_body(h_idx, r_idx, t_idx, ent_hbm, rel_hbm,
                 h_out, r_out, t_out,
                 idx_v, rows_v, sem):
    wid = lax.axis_index("s") * NUM_CORES + lax.axis_index("c")
    row0 = wid * CPW

    for tbl_i, (idx_hbm, table, out) in enumerate((
            (h_idx, ent_hbm, h_out),
            (r_idx, rel_hbm, r_out),
            (t_idx, ent_hbm, t_out))):
        pltpu.sync_copy(idx_hbm.at[pl.ds(row0, CPW)], idx_v.at[tbl_i])
        for j in range(CPW):
            pltpu.async_copy(table.at[idx_v.at[tbl_i].at[j]], rows_v, sem).wait()
            pltpu.sync_copy(rows_v, out.at[pl.ds((row0 + j) * CHUNK, CHUNK)])


def _transe(h2, r2, t2, entity_emb, relation_emb):
    mesh = plsc.VectorSubcoreMesh(core_axis_name="c", subcore_axis_name="s")
    out_t = (jax.ShapeDtypeStruct((B, D), jnp.float32),) * 3
    run = functools.partial(
        pl.kernel, mesh=mesh,
        out_type=out_t,
        scratch_types=[
            pltpu.VMEM((3, CPW, CHUNK), jnp.int32),
            pltpu.VMEM((CHUNK, D), jnp.float32),
            pltpu.SemaphoreType.DMA,
        ],
    )(_transe_body)
    return run(h2, r2, t2, entity_emb, relation_emb)


def kernel(h, r, t, entity_emb, relation_emb):
    h2 = h.astype(jnp.int32).reshape(B // CHUNK, CHUNK)
    r2 = r.astype(jnp.int32).reshape(B // CHUNK, CHUNK)
    t2 = t.astype(jnp.int32).reshape(B // CHUNK, CHUNK)
    return _transe(h2, r2, t2, entity_emb, relation_emb)


# speedup vs baseline: 2.6491x; 1.2191x over previous
import functools

import jax
import jax.numpy as jnp
from jax import lax
from jax.experimental import pallas as pl
from jax.experimental.pallas import tpu as pltpu
from jax.experimental.pallas import tpu_sc as plsc

NUM_CORES = 2
NUM_SUBCORES = 16
NW = NUM_CORES * NUM_SUBCORES
B = 16384
D = 128
CHUNK = 128
CPW = B // (NW * CHUNK)


NBUF = 6
NCH = 3 * CPW


def _transe_body(h_idx, r_idx, t_idx, ent_hbm, rel_hbm,
                 h_out, r_out, t_out,
                 idx_v, bufs, gsem, wsem):
    wid = lax.axis_index("s") * NUM_CORES + lax.axis_index("c")
    row0 = wid * CPW

    for tbl_i, idx_hbm in enumerate((h_idx, r_idx, t_idx)):
        pltpu.sync_copy(idx_hbm.at[pl.ds(row0, CPW)], idx_v.at[tbl_i])

    tables = (ent_hbm, rel_hbm, ent_hbm)
    outs = (h_out, r_out, t_out)

    def start_gather(c):
        tbl_i, j = divmod(c, CPW)
        return pltpu.async_copy(
            tables[tbl_i].at[idx_v.at[tbl_i].at[j]], bufs.at[c % NBUF], gsem)

    def start_wb(c):
        tbl_i, j = divmod(c, CPW)
        return pltpu.async_copy(
            bufs.at[c % NBUF], outs[tbl_i].at[pl.ds((row0 + j) * CHUNK, CHUNK)], wsem)

    g = [None] * NCH
    w = [None] * NCH
    for c in range(NBUF):
        g[c] = start_gather(c)
    for c in range(NCH):
        if 0 < c <= NCH - NBUF:
            w[c - 1].wait()
            g[c - 1 + NBUF] = start_gather(c - 1 + NBUF)
        g[c].wait()
        w[c] = start_wb(c)
    for c in range(max(0, NCH - NBUF), NCH):
        w[c].wait()


def _transe(h2, r2, t2, entity_emb, relation_emb):
    mesh = plsc.VectorSubcoreMesh(core_axis_name="c", subcore_axis_name="s")
    out_t = (jax.ShapeDtypeStruct((B, D), jnp.float32),) * 3
    run = functools.partial(
        pl.kernel, mesh=mesh,
        out_type=out_t,
        scratch_types=[
            pltpu.VMEM((3, CPW, CHUNK), jnp.int32),
            pltpu.VMEM((NBUF, CHUNK, D), jnp.float32),
            pltpu.SemaphoreType.DMA,
            pltpu.SemaphoreType.DMA,
        ],
    )(_transe_body)
    return run(h2, r2, t2, entity_emb, relation_emb)


def kernel(h, r, t, entity_emb, relation_emb):
    h2 = h.astype(jnp.int32).reshape(B // CHUNK, CHUNK)
    r2 = r.astype(jnp.int32).reshape(B // CHUNK, CHUNK)
    t2 = t.astype(jnp.int32).reshape(B // CHUNK, CHUNK)
    return _transe(h2, r2, t2, entity_emb, relation_emb)
